# baseline (speedup 1.0000x reference)
import functools
import jax
import jax.numpy as jnp
from jax import lax
from jax.experimental import pallas as pl
from jax.experimental.pallas import tpu as pltpu
from jax.experimental.pallas import tpu_sc as plsc

_N = 1000
_D = 1024
_R = 16
_K = 50
_BETA2 = 0.2 ** 2
_TAU = 0.01
_SCALING = 2.0
_KSC = 64
_ROW = _R * _D


def _routing_body(q_ref, cor_ref, idx_ref, sc_ref):
    q = q_ref[...]
    qn = jnp.sqrt(jnp.sum(q * q))
    ii = lax.broadcasted_iota(jnp.int32, (_N, _N), 0)
    jj = lax.broadcasted_iota(jnp.int32, (_N, _N), 1)
    eye_n = jnp.where(ii == jj, 1.0, 0.0)
    scores = lax.dot_general(q, cor_ref[...], (((1,), (1,)), ((), ())),
                             preferred_element_type=jnp.float32)
    csq = jnp.zeros((1, _N), jnp.float32)
    ones = jnp.ones((1, 128), jnp.float32)
    for t in range(_D // 128):
        ch = cor_ref[:, 128 * t:128 * (t + 1)]
        csq = csq + lax.dot_general(ones, ch * ch, (((1,), (1,)), ((), ())),
                                    preferred_element_type=jnp.float32)
    sim = scores / ((jnp.sqrt(csq) + 1e-9) * (qn + 1e-9)) / _BETA2
    mx = jnp.max(sim)
    e = jnp.exp(sim - mx)
    p = e / jnp.sum(e)
    p = jnp.where(p >= _TAU, p, 0.0)
    p_col = lax.dot_general(eye_n, p, (((1,), (1,)), ((), ())),
                            preferred_element_type=jnp.float32)
    cmp = jnp.where(p_col < p, 1.0, 0.0)
    rank = lax.dot_general(cmp, jnp.ones((_N, 1), jnp.float32),
                           (((1,), (0,)), ((), ())),
                           preferred_element_type=jnp.float32)
    lane64 = lax.broadcasted_iota(jnp.int32, (1, _KSC), 1).astype(jnp.float32)
    onehot = jnp.where(rank == lane64, 1.0, 0.0)
    lane_f = lax.broadcasted_iota(jnp.int32, (1, _N), 1).astype(jnp.float32)
    idx_f = lax.dot_general(lane_f, onehot, (((1,), (0,)), ((), ())),
                            preferred_element_type=jnp.float32)
    wvec = lax.dot_general(p, onehot, (((1,), (0,)), ((), ())),
                           preferred_element_type=jnp.float32)
    sel = lane64 < float(_K)
    ssum = jnp.sum(jnp.where(sel, wvec, 0.0))
    wscale = _SCALING / (ssum + 1e-9)
    sc_ref[...] = jnp.where(sel, wvec * wscale, 0.0)
    pr = lax.broadcasted_iota(jnp.int32, (_KSC, 256), 0)
    pc = lax.broadcasted_iota(jnp.int32, (_KSC, 256), 1)
    perm = jnp.where(pc == 8 * (pr // 2) + pr % 2, 1.0, 0.0)
    idx256 = lax.dot_general(idx_f, perm, (((1,), (0,)), ((), ())),
                             preferred_element_type=jnp.float32)
    idx_ref[...] = jnp.clip(idx256, 0.0, float(_N - 1)).astype(jnp.int32)


def _routing(q, corpus):
    return pl.pallas_call(
        _routing_body,
        out_shape=(jax.ShapeDtypeStruct((1, 256), jnp.int32),
                   jax.ShapeDtypeStruct((1, _KSC), jnp.float32)),
        in_specs=[pl.BlockSpec(memory_space=pltpu.VMEM),
                  pl.BlockSpec(memory_space=pltpu.VMEM)],
        out_specs=(pl.BlockSpec(memory_space=pltpu.VMEM),
                   pl.BlockSpec(memory_space=pltpu.VMEM)),
    )(q, corpus)


def _sc_gather(A2, Bt2, idx64):
    mesh = plsc.VectorSubcoreMesh(core_axis_name="c", subcore_axis_name="s")

    @functools.partial(
        pl.kernel, mesh=mesh,
        out_type=(jax.ShapeDtypeStruct((_KSC, _ROW), jnp.float32),
                  jax.ShapeDtypeStruct((_KSC, _ROW), jnp.float32)),
        scratch_types=[
            pltpu.VMEM((8,), jnp.int32),
            pltpu.VMEM((2, _ROW), jnp.float32),
            pltpu.SemaphoreType.DMA,
        ],
    )
    def k(a_hbm, b_hbm, idx_hbm, aout, bout, idx_v, rows_v, sem):
        cid = lax.axis_index("c")
        sid = lax.axis_index("s")
        wid = sid * 2 + cid
        pltpu.sync_copy(idx_hbm.at[wid], idx_v)
        my = idx_v.at[pl.ds(0, 2)]
        pltpu.async_copy(a_hbm.at[my], rows_v, sem).wait()
        pltpu.sync_copy(rows_v, aout.at[pl.ds(wid * 2, 2)])
        pltpu.async_copy(b_hbm.at[my], rows_v, sem).wait()
        pltpu.sync_copy(rows_v, bout.at[pl.ds(wid * 2, 2)])

    return k(A2, Bt2, idx64)


def _merge_body(aa_ref, bb_ref, wb_ref, sc_ref, out_ref):
    rowg = lax.broadcasted_iota(jnp.int32, (_KSC * _R, _KSC), 0) // _R
    kcol = lax.broadcasted_iota(jnp.int32, (_KSC * _R, _KSC), 1)
    eye_g = jnp.where(rowg == kcol, 1.0, 0.0)
    scale_col = lax.dot_general(eye_g, sc_ref[...], (((1,), (1,)), ((), ())),
                                preferred_element_type=jnp.float32)
    delta = lax.dot_general(
        (bb_ref[...] * scale_col).astype(jnp.bfloat16),
        aa_ref[...].astype(jnp.bfloat16),
        (((0,), (0,)), ((), ())), preferred_element_type=jnp.float32)
    out_ref[...] = wb_ref[...] + delta


def _merge(aa, bb, W_base, sc64):
    return pl.pallas_call(
        _merge_body,
        out_shape=jax.ShapeDtypeStruct((_D, _D), jnp.float32),
        in_specs=[pl.BlockSpec(memory_space=pltpu.VMEM),
                  pl.BlockSpec(memory_space=pltpu.VMEM),
                  pl.BlockSpec(memory_space=pltpu.VMEM),
                  pl.BlockSpec(memory_space=pltpu.VMEM)],
        out_specs=pl.BlockSpec(memory_space=pltpu.VMEM),
    )(aa, bb, W_base, sc64)


def kernel(q, corpus, A_all, B_all, W_base):
    B_t = jnp.swapaxes(B_all, 1, 2)
    idx64, sc64 = _routing(q, corpus)
    A2 = A_all.reshape(_N, _ROW)
    Bt2 = B_t.reshape(_N, _ROW)
    aout, bout = _sc_gather(A2, Bt2, idx64.reshape(32, 8))
    aa = aout.reshape(_KSC * _R, _D)
    bb = bout.reshape(_KSC * _R, _D)
    return _merge(aa, bb, W_base, sc64)
